# baseline (speedup 1.0000x reference)
import jax
import jax.numpy as jnp
from jax.experimental import pallas as pl

_NB = 16


def _lsh_vote_kernel(x_ref, w_ref, out_ref):
    x = x_ref[...]
    w = w_ref[...]
    proj = jnp.dot(x, w, preferred_element_type=jnp.float32)
    bits = (proj > 0).astype(jnp.float32)
    c_total = w.shape[1]
    v_total = c_total // _NB
    c = jax.lax.broadcasted_iota(jnp.int32, (c_total, v_total), 0)
    v = jax.lax.broadcasted_iota(jnp.int32, (c_total, v_total), 1)
    pow2 = jnp.left_shift(jnp.int32(1), c % _NB).astype(jnp.float32)
    pack = jnp.where(c // _NB == v, pow2, 0.0)
    votes = jnp.dot(bits, pack, preferred_element_type=jnp.float32)
    out_ref[...] = votes.astype(jnp.int32)


def kernel(x, W):
    Q, D = x.shape
    V, _, B = W.shape
    W2 = jnp.transpose(W, (1, 0, 2)).reshape(D, V * B)
    BQ = 1024
    out = pl.pallas_call(
        _lsh_vote_kernel,
        grid=(Q // BQ,),
        in_specs=[
            pl.BlockSpec((BQ, D), lambda i: (i, 0)),
            pl.BlockSpec((D, V * B), lambda i: (0, 0)),
        ],
        out_specs=pl.BlockSpec((BQ, V), lambda i: (i, 0)),
        out_shape=jax.ShapeDtypeStruct((Q, V), jnp.int32),
    )(x, W2)
    return out.T


# speedup vs baseline: 1.4769x; 1.4769x over previous
import jax
import jax.numpy as jnp
from jax.experimental import pallas as pl
from jax.experimental.pallas import tpu as pltpu

_NB = 16


def _lsh_vote_kernel(x_ref, w_ref, out_ref):
    x = x_ref[...]
    w = w_ref[...]
    proj = jnp.dot(x, w, preferred_element_type=jnp.float32)
    bits = (proj > 0).astype(jnp.float32)
    c_total = w.shape[1]
    v_total = c_total // _NB
    c = jax.lax.broadcasted_iota(jnp.int32, (v_total, c_total), 1)
    v = jax.lax.broadcasted_iota(jnp.int32, (v_total, c_total), 0)
    pow2 = jnp.left_shift(jnp.int32(1), c % _NB).astype(jnp.float32)
    packT = jnp.where(c // _NB == v, pow2, 0.0)
    votes_t = jax.lax.dot_general(
        packT, bits, (((1,), (1,)), ((), ())),
        preferred_element_type=jnp.float32)
    out_ref[...] = votes_t.astype(jnp.int32)


def kernel(x, W):
    Q, D = x.shape
    V, _, B = W.shape
    W2 = jnp.transpose(W, (1, 0, 2)).reshape(D, V * B)
    BQ = 1024
    return pl.pallas_call(
        _lsh_vote_kernel,
        grid=(Q // BQ,),
        in_specs=[
            pl.BlockSpec((BQ, D), lambda i: (i, 0)),
            pl.BlockSpec((D, V * B), lambda i: (0, 0)),
        ],
        out_specs=pl.BlockSpec((V, BQ), lambda i: (0, i)),
        out_shape=jax.ShapeDtypeStruct((V, Q), jnp.int32),
        compiler_params=pltpu.CompilerParams(
            dimension_semantics=("parallel",)),
    )(x, W2)


# speedup vs baseline: 2.0172x; 1.3658x over previous
import jax
import jax.numpy as jnp
from jax.experimental import pallas as pl
from jax.experimental.pallas import tpu as pltpu

_NB = 16


def _lsh_vote_kernel(x_ref, w_ref, out_ref):
    x = x_ref[...]
    w = w_ref[...]
    proj = jnp.dot(x, w, preferred_element_type=jnp.float32)
    bits = (proj > 0).astype(jnp.bfloat16)
    c_total = w.shape[1]
    v_total = c_total // _NB
    c = jax.lax.broadcasted_iota(jnp.int32, (v_total, c_total), 1)
    v = jax.lax.broadcasted_iota(jnp.int32, (v_total, c_total), 0)
    pow2 = jnp.left_shift(jnp.int32(1), c % _NB).astype(jnp.float32)
    packT = jnp.where(c // _NB == v, pow2, 0.0).astype(jnp.bfloat16)
    votes_t = jax.lax.dot_general(
        packT, bits, (((1,), (1,)), ((), ())),
        preferred_element_type=jnp.float32)
    out_ref[...] = votes_t.astype(jnp.int32)


def kernel(x, W):
    Q, D = x.shape
    V, _, B = W.shape
    W2 = jnp.transpose(W, (1, 0, 2)).reshape(D, V * B)
    BQ = 2048
    return pl.pallas_call(
        _lsh_vote_kernel,
        grid=(Q // BQ,),
        in_specs=[
            pl.BlockSpec((BQ, D), lambda i: (i, 0)),
            pl.BlockSpec((D, V * B), lambda i: (0, 0)),
        ],
        out_specs=pl.BlockSpec((V, BQ), lambda i: (0, i)),
        out_shape=jax.ShapeDtypeStruct((V, Q), jnp.int32),
        compiler_params=pltpu.CompilerParams(
            dimension_semantics=("parallel",)),
    )(x, W2)


# speedup vs baseline: 2.3951x; 1.1874x over previous
import jax
import jax.numpy as jnp
from jax.experimental import pallas as pl
from jax.experimental.pallas import tpu as pltpu

_NB = 16


def _lsh_vote_kernel(x_ref, w_ref, out_ref):
    x = x_ref[...]
    w = w_ref[...]
    proj = jnp.dot(x, w, preferred_element_type=jnp.float32)
    bits = (proj > 0).astype(jnp.bfloat16)
    c_total = w.shape[1]
    v_total = c_total // _NB
    c = jax.lax.broadcasted_iota(jnp.int32, (v_total, c_total), 1)
    v = jax.lax.broadcasted_iota(jnp.int32, (v_total, c_total), 0)
    pow2 = jnp.left_shift(jnp.int32(1), c % _NB).astype(jnp.float32)
    packT = jnp.where(c // _NB == v, pow2, 0.0).astype(jnp.bfloat16)
    votes_t = jax.lax.dot_general(
        packT, bits, (((1,), (1,)), ((), ())),
        preferred_element_type=jnp.float32)
    out_ref[...] = votes_t.astype(jnp.int32)


def kernel(x, W):
    Q, D = x.shape
    V, _, B = W.shape
    W2 = jnp.transpose(W, (1, 0, 2)).reshape(D, V * B)
    BQ = 4096
    return pl.pallas_call(
        _lsh_vote_kernel,
        grid=(Q // BQ,),
        in_specs=[
            pl.BlockSpec((BQ, D), lambda i: (i, 0)),
            pl.BlockSpec((D, V * B), lambda i: (0, 0)),
        ],
        out_specs=pl.BlockSpec((V, BQ), lambda i: (0, i)),
        out_shape=jax.ShapeDtypeStruct((V, Q), jnp.int32),
        compiler_params=pltpu.CompilerParams(
            dimension_semantics=("parallel",)),
    )(x, W2)


# speedup vs baseline: 2.4570x; 1.0259x over previous
import jax
import jax.numpy as jnp
from jax.experimental import pallas as pl
from jax.experimental.pallas import tpu as pltpu

_NB = 16


def _lsh_vote_kernel(x_ref, w_ref, out_ref):
    x = x_ref[...]
    w = w_ref[...]
    proj = jnp.dot(x, w, preferred_element_type=jnp.float32)
    bits = (proj > 0).astype(jnp.bfloat16)
    c_total = w.shape[1]
    v_total = c_total // _NB
    c = jax.lax.broadcasted_iota(jnp.int32, (v_total, c_total), 1)
    v = jax.lax.broadcasted_iota(jnp.int32, (v_total, c_total), 0)
    pow2 = jnp.left_shift(jnp.int32(1), c % _NB).astype(jnp.float32)
    packT = jnp.where(c // _NB == v, pow2, 0.0).astype(jnp.bfloat16)
    votes_t = jax.lax.dot_general(
        packT, bits, (((1,), (1,)), ((), ())),
        preferred_element_type=jnp.float32)
    out_ref[...] = votes_t.astype(jnp.int32)


def kernel(x, W):
    Q, D = x.shape
    V, _, B = W.shape
    W2 = jnp.transpose(W, (1, 0, 2)).reshape(D, V * B)
    BQ = 8192
    return pl.pallas_call(
        _lsh_vote_kernel,
        grid=(Q // BQ,),
        in_specs=[
            pl.BlockSpec((BQ, D), lambda i: (i, 0)),
            pl.BlockSpec((D, V * B), lambda i: (0, 0)),
        ],
        out_specs=pl.BlockSpec((V, BQ), lambda i: (0, i)),
        out_shape=jax.ShapeDtypeStruct((V, Q), jnp.int32),
        compiler_params=pltpu.CompilerParams(
            dimension_semantics=("parallel",)),
    )(x, W2)
